# baseline (speedup 1.0000x reference)
import jax
import jax.numpy as jnp
from jax.experimental import pallas as pl
from jax.experimental.pallas import tpu as pltpu

_NB = 2


def _body(img_ref, msk_ref, w_ref, b_ref, out_ref):
    C = 3
    H = img_ref.shape[1] // C
    Wd = img_ref.shape[2]
    CH = 16
    inv = 1.0 / (H * Wd)

    def red(c):
        def body(k, acc):
            chunk = img_ref[:, pl.ds(c * H + k * CH, CH), :]
            return acc + jnp.sum(chunk.reshape(_NB, CH // 8, 8, Wd), axis=1)
        acc = jax.lax.fori_loop(0, H // CH, body, jnp.zeros((_NB, 8, Wd), jnp.float32))
        return jnp.sum(acc, axis=(1, 2)) * inv
    pooled = jnp.stack([red(c) for c in range(C)], axis=1)

    logits = jnp.sum(pooled[:, :, None] * w_ref[...][None], axis=1) + b_ref[...][None]
    sig = jax.nn.sigmoid(logits)
    t = sig[:, 3][:, None, None]
    cb = (sig[:, :3] * (1.0 - sig[:, 3:4]))

    def ew(c):
        cbc = cb[:, c][:, None, None]
        def body(k, carry):
            sl = pl.ds(k * CH, CH)
            x = img_ref[:, pl.ds(c * H + k * CH, CH), :]
            m = msk_ref[:, sl, :] != 0
            out_ref[:, pl.ds(c * H + k * CH, CH), :] = jnp.where(m, cbc + t * x, x)
            return carry
        jax.lax.fori_loop(0, H // CH, body, 0)
    for c in range(C):
        ew(c)


def kernel(image, mask, W, b):
    B, C, H, Wd = image.shape
    img2 = image.reshape(B, C * H, Wd)
    out = pl.pallas_call(
        _body,
        grid=(B // _NB,),
        in_specs=[
            pl.BlockSpec((_NB, C * H, Wd), lambda i: (i, 0, 0)),
            pl.BlockSpec((_NB, H, Wd), lambda i: (i, 0, 0)),
            pl.BlockSpec((C, 4), lambda i: (0, 0)),
            pl.BlockSpec((4,), lambda i: (0,)),
        ],
        out_specs=pl.BlockSpec((_NB, C * H, Wd), lambda i: (i, 0, 0)),
        out_shape=jax.ShapeDtypeStruct(img2.shape, image.dtype),
        compiler_params=pltpu.CompilerParams(
            dimension_semantics=("arbitrary",),
            vmem_limit_bytes=100 * 1024 * 1024,
        ),
    )(img2, mask, W, b)
    return out.reshape(image.shape)


# speedup vs baseline: 2.3458x; 2.3458x over previous
import jax
import jax.numpy as jnp
from jax.experimental import pallas as pl
from jax.experimental.pallas import tpu as pltpu

_NB = 2


def _body(img_ref, msk_ref, w_ref, b_ref, out_ref):
    C = 3
    H = img_ref.shape[1] // C
    Wd = img_ref.shape[2]
    CH = 128
    inv = 1.0 / (H * Wd)

    def red(c):
        def body(k, acc):
            chunk = img_ref[:, pl.ds(c * H + k * CH, CH), :]
            return acc + jnp.sum(chunk.reshape(_NB, CH // 8, 8, Wd), axis=1)
        acc = jax.lax.fori_loop(0, H // CH, body, jnp.zeros((_NB, 8, Wd), jnp.float32))
        return jnp.sum(acc, axis=(1, 2)) * inv
    pooled = jnp.stack([red(c) for c in range(C)], axis=1)

    logits = jnp.sum(pooled[:, :, None] * w_ref[...][None], axis=1) + b_ref[...][None]
    sig = jax.nn.sigmoid(logits)
    t = sig[:, 3][:, None, None]
    cb = (sig[:, :3] * (1.0 - sig[:, 3:4]))

    def ew(c):
        cbc = cb[:, c][:, None, None]
        def body(k, carry):
            sl = pl.ds(k * CH, CH)
            x = img_ref[:, pl.ds(c * H + k * CH, CH), :]
            m = msk_ref[:, sl, :] != 0
            out_ref[:, pl.ds(c * H + k * CH, CH), :] = jnp.where(m, cbc + t * x, x)
            return carry
        jax.lax.fori_loop(0, H // CH, body, 0)
    for c in range(C):
        ew(c)


def kernel(image, mask, W, b):
    B, C, H, Wd = image.shape
    img2 = image.reshape(B, C * H, Wd)
    out = pl.pallas_call(
        _body,
        grid=(B // _NB,),
        in_specs=[
            pl.BlockSpec((_NB, C * H, Wd), lambda i: (i, 0, 0)),
            pl.BlockSpec((_NB, H, Wd), lambda i: (i, 0, 0)),
            pl.BlockSpec((C, 4), lambda i: (0, 0)),
            pl.BlockSpec((4,), lambda i: (0,)),
        ],
        out_specs=pl.BlockSpec((_NB, C * H, Wd), lambda i: (i, 0, 0)),
        out_shape=jax.ShapeDtypeStruct(img2.shape, image.dtype),
        compiler_params=pltpu.CompilerParams(
            dimension_semantics=("arbitrary",),
            vmem_limit_bytes=100 * 1024 * 1024,
        ),
    )(img2, mask, W, b)
    return out.reshape(image.shape)


# speedup vs baseline: 2.5839x; 1.1015x over previous
import jax
import jax.numpy as jnp
from jax.experimental import pallas as pl
from jax.experimental.pallas import tpu as pltpu

_NB = 2


def _body(img_ref, msk_ref, w_ref, b_ref, out_ref):
    C = 3
    H = img_ref.shape[1] // C
    pooled = jnp.mean(img_ref[...].reshape(_NB, C, H, -1), axis=(2, 3))
    logits = jnp.sum(pooled[:, :, None] * w_ref[...][None], axis=1) + b_ref[...][None]
    sig = jax.nn.sigmoid(logits)
    color = sig[:, :3]
    t = sig[:, 3][:, None, None, None]
    cb = color[:, :, None, None] * (1.0 - t)
    m = (msk_ref[...] != 0)[:, None, :, :]
    x4 = img_ref[...].reshape(_NB, C, H, -1)
    out_ref[...] = jnp.where(m, cb + t * x4, x4).reshape(img_ref.shape)


def kernel(image, mask, W, b):
    B, C, H, Wd = image.shape
    img2 = image.reshape(B, C * H, Wd)
    out = pl.pallas_call(
        _body,
        grid=(B // _NB,),
        in_specs=[
            pl.BlockSpec((_NB, C * H, Wd), lambda i: (i, 0, 0)),
            pl.BlockSpec((_NB, H, Wd), lambda i: (i, 0, 0)),
            pl.BlockSpec((C, 4), lambda i: (0, 0)),
            pl.BlockSpec((4,), lambda i: (0,)),
        ],
        out_specs=pl.BlockSpec((_NB, C * H, Wd), lambda i: (i, 0, 0)),
        out_shape=jax.ShapeDtypeStruct(img2.shape, image.dtype),
        compiler_params=pltpu.CompilerParams(
            dimension_semantics=("arbitrary",),
            vmem_limit_bytes=100 * 1024 * 1024,
        ),
    )(img2, mask, W, b)
    return out.reshape(image.shape)


# speedup vs baseline: 2.5975x; 1.0053x over previous
import jax
import jax.numpy as jnp
from jax.experimental import pallas as pl
from jax.experimental.pallas import tpu as pltpu

_NB = 2


def _body(img_ref, msk_ref, w_ref, b_ref, out_ref):
    C = 3
    H = img_ref.shape[1] // C
    s1 = jnp.sum(img_ref[...].reshape(_NB, C, H, -1), axis=2)
    pooled = jnp.sum(s1, axis=-1) * (1.0 / (H * img_ref.shape[2]))
    logits = jnp.sum(pooled[:, :, None] * w_ref[...][None], axis=1) + b_ref[...][None]
    sig = jax.nn.sigmoid(logits)
    t = sig[:, 3][:, None, None]
    cb = sig[:, :3] * (1.0 - sig[:, 3:4])
    for c in range(C):
        x = img_ref[:, pl.ds(c * H, H), :]
        m = msk_ref[...] != 0
        out_ref[:, pl.ds(c * H, H), :] = jnp.where(
            m, cb[:, c][:, None, None] + t * x, x)


def kernel(image, mask, W, b):
    B, C, H, Wd = image.shape
    img2 = image.reshape(B, C * H, Wd)
    out = pl.pallas_call(
        _body,
        grid=(B // _NB,),
        in_specs=[
            pl.BlockSpec((_NB, C * H, Wd), lambda i: (i, 0, 0)),
            pl.BlockSpec((_NB, H, Wd), lambda i: (i, 0, 0)),
            pl.BlockSpec((C, 4), lambda i: (0, 0)),
            pl.BlockSpec((4,), lambda i: (0,)),
        ],
        out_specs=pl.BlockSpec((_NB, C * H, Wd), lambda i: (i, 0, 0)),
        out_shape=jax.ShapeDtypeStruct(img2.shape, image.dtype),
        compiler_params=pltpu.CompilerParams(
            dimension_semantics=("arbitrary",),
            vmem_limit_bytes=100 * 1024 * 1024,
        ),
    )(img2, mask, W, b)
    return out.reshape(image.shape)
